# Initial kernel scaffold; baseline (speedup 1.0000x reference)
import jax
import jax.numpy as jnp
from jax.experimental import pallas as pl


def kernel(x, edge_index, W1, W2, W3, W4):
    raise NotImplementedError("write your pallas kernel here")


# speedup vs baseline: 4.6831x; 4.6831x over previous
import functools

import jax
import jax.numpy as jnp
from jax import lax
from jax.experimental import pallas as pl
from jax.experimental.pallas import tpu as pltpu
from jax.experimental.pallas import tpu_sc as plsc

NC = 2
NS = 16
NW = NC * NS
CH = 128


def _spmm_partials(support, src, dst, n_rows):
  e_pad = src.shape[0]
  h = support.shape[1]
  ept = e_pad // NW
  n_chunks = ept // CH
  acc_rows = -(-(n_rows + 1) // (NS * CH)) * (NS * CH)
  zpt = acc_rows // (NS * CH)
  rpt = ((n_rows + NS - 1) // NS + 7) // 8 * 8
  rpt_last = n_rows - rpt * (NS - 1)
  assert rpt_last > 0

  mesh = plsc.VectorSubcoreMesh(core_axis_name="c", subcore_axis_name="s")

  @functools.partial(
      pl.kernel,
      out_type=jax.ShapeDtypeStruct((NC, n_rows, h), jnp.float32),
      mesh=mesh,
      scratch_types=[
          pltpu.VMEM((CH,), jnp.int32),
          pltpu.VMEM((CH,), jnp.int32),
          pltpu.VMEM((CH, h), jnp.float32),
          pltpu.VMEM_SHARED((acc_rows, h), jnp.float32),
          pltpu.SemaphoreType.DMA,
      ],
      compiler_params=pltpu.CompilerParams(use_tc_tiling_on_sc=False),
  )
  def spmm(sup_hbm, src_hbm, dst_hbm, out_hbm, sidx, didx, rows, acc, sem):
    cid = lax.axis_index("c")
    sid = lax.axis_index("s")
    wid = sid * NC + cid

    def zrow(j, carry):
      for k in range(h // 16):
        rows[j, pl.ds(k * 16, 16)] = jnp.zeros((16,), jnp.float32)
      return carry
    lax.fori_loop(0, CH, zrow, 0)
    for z in range(zpt):
      pltpu.sync_copy(rows, acc.at[pl.ds((sid * zpt + z) * CH, CH)])
    plsc.subcore_barrier()

    ebase = wid * ept

    def body(g, carry):
      off = ebase + g * CH
      pltpu.sync_copy(src_hbm.at[pl.ds(off, CH)], sidx)
      pltpu.sync_copy(dst_hbm.at[pl.ds(off, CH)], didx)
      pltpu.async_copy(sup_hbm.at[sidx], rows, sem).wait()
      pltpu.sync_copy(rows, acc.at[didx], add=True)
      return carry
    lax.fori_loop(0, n_chunks, body, 0)

    plsc.subcore_barrier()

    @pl.when(sid < NS - 1)
    def _():
      pltpu.sync_copy(acc.at[pl.ds(sid * rpt, rpt)],
                      out_hbm.at[cid].at[pl.ds(sid * rpt, rpt)])

    @pl.when(sid == NS - 1)
    def _():
      pltpu.sync_copy(acc.at[pl.ds((NS - 1) * rpt, rpt_last)],
                      out_hbm.at[cid].at[pl.ds((NS - 1) * rpt, rpt_last)])

  return spmm(support, src, dst)


def _mm(x, w, bm):
  n, d = x.shape
  h = w.shape[1]

  def body(x_ref, w_ref, o_ref):
    o_ref[...] = lax.dot_general(
        x_ref[...], w_ref[...], (((1,), (0,)), ((), ())),
        preferred_element_type=jnp.float32, precision=lax.Precision.HIGHEST)

  return pl.pallas_call(
      body,
      grid=(n // bm,),
      in_specs=[
          pl.BlockSpec((bm, d), lambda i: (i, 0)),
          pl.BlockSpec((d, h), lambda i: (0, 0)),
      ],
      out_specs=pl.BlockSpec((bm, h), lambda i: (i, 0)),
      out_shape=jax.ShapeDtypeStruct((n, h), jnp.float32),
  )(x, w)


def _fused_relu_mm(p, w, bm):
  _, n, d = p.shape
  h = w.shape[1]

  def body(p_ref, w_ref, o_ref):
    hid = jnp.maximum(p_ref[0] + p_ref[1], 0.0)
    o_ref[...] = lax.dot_general(
        hid, w_ref[...], (((1,), (0,)), ((), ())),
        preferred_element_type=jnp.float32, precision=lax.Precision.HIGHEST)

  return pl.pallas_call(
      body,
      grid=(n // bm,),
      in_specs=[
          pl.BlockSpec((2, bm, d), lambda i: (0, i, 0)),
          pl.BlockSpec((d, h), lambda i: (0, 0)),
      ],
      out_specs=pl.BlockSpec((bm, h), lambda i: (i, 0)),
      out_shape=jax.ShapeDtypeStruct((n, h), jnp.float32),
  )(p, w)


def _decoder(p34, h3, bm, bn):
  _, n, h2 = p34.shape

  def body(pi_ref, pj_ref, dc_ref, mu_ref, lv_ref):
    zi_full = pi_ref[0] + pi_ref[1]
    zj_full = pj_ref[0] + pj_ref[1]
    zi = zi_full[:, :h3]
    zj = zj_full[:, :h3]
    dc_ref[...] = lax.dot_general(
        zi, zj, (((1,), (1,)), ((), ())),
        preferred_element_type=jnp.float32, precision=lax.Precision.HIGHEST)

    @pl.when(pl.program_id(1) == 0)
    def _():
      mu_ref[...] = zi
      lv_ref[...] = zi_full[:, h3:]

  return pl.pallas_call(
      body,
      grid=(-(-n // bm), -(-n // bn)),
      in_specs=[
          pl.BlockSpec((2, bm, h2), lambda i, j: (0, i, 0)),
          pl.BlockSpec((2, bn, h2), lambda i, j: (0, j, 0)),
      ],
      out_specs=[
          pl.BlockSpec((bm, bn), lambda i, j: (i, j)),
          pl.BlockSpec((bm, h3), lambda i, j: (i, 0)),
          pl.BlockSpec((bm, h3), lambda i, j: (i, 0)),
      ],
      out_shape=[
          jax.ShapeDtypeStruct((n, n), jnp.float32),
          jax.ShapeDtypeStruct((n, h3), jnp.float32),
          jax.ShapeDtypeStruct((n, h3), jnp.float32),
      ],
  )(p34, p34)


@jax.jit
def kernel(x, edge_index, W1, W2, W3, W4):
  n, _ = x.shape
  e = edge_index.shape[1]

  src = edge_index[0].astype(jnp.int32)
  dst = edge_index[1].astype(jnp.int32)
  e_pad = -(-e // (NW * CH)) * (NW * CH)
  if e_pad != e:
    pad = e_pad - e
    src = jnp.concatenate([src, jnp.zeros((pad,), jnp.int32)])
    dst = jnp.concatenate([dst, jnp.full((pad,), n, jnp.int32)])

  support1 = _mm(x, W1, bm=2000)
  p1 = _spmm_partials(support1, src, dst, n)
  support2 = _fused_relu_mm(p1, W2, bm=2000)
  p2 = _spmm_partials(support2, src, dst, n)
  w34 = jnp.concatenate([W3, W4], axis=1)
  support34 = _fused_relu_mm(p2, w34, bm=2000)
  p34 = _spmm_partials(support34, src, dst, n)
  dc, mu, logvar = _decoder(p34, W3.shape[1], bm=1000, bn=2048)
  return (dc, mu, logvar)


# speedup vs baseline: 4.8877x; 1.0437x over previous
import functools

import jax
import jax.numpy as jnp
from jax import lax
from jax.experimental import pallas as pl
from jax.experimental.pallas import tpu as pltpu
from jax.experimental.pallas import tpu_sc as plsc

NC = 2
NS = 16
NW = NC * NS
CH = 128


def _spmm_partials(support, src, dst, n_rows):
  e_pad = src.shape[0]
  h = support.shape[1]
  ept = e_pad // NW
  n_chunks = ept // CH
  acc_rows = -(-(n_rows + 1) // (NS * CH)) * (NS * CH)
  zpt = acc_rows // (NS * CH)
  rpt = ((n_rows + NS - 1) // NS + 7) // 8 * 8
  rpt_last = n_rows - rpt * (NS - 1)
  assert rpt_last > 0

  mesh = plsc.VectorSubcoreMesh(core_axis_name="c", subcore_axis_name="s")

  assert n_chunks % 2 == 0
  n_pairs = n_chunks // 2

  @functools.partial(
      pl.kernel,
      out_type=jax.ShapeDtypeStruct((NC, n_rows, h), jnp.float32),
      mesh=mesh,
      scratch_types=[
          pltpu.VMEM((2, CH), jnp.int32),
          pltpu.VMEM((2, CH), jnp.int32),
          pltpu.VMEM((2, CH, h), jnp.float32),
          pltpu.VMEM_SHARED((acc_rows, h), jnp.float32),
          pltpu.SemaphoreType.DMA,
          pltpu.SemaphoreType.DMA,
          pltpu.SemaphoreType.DMA,
          pltpu.SemaphoreType.DMA,
      ],
      compiler_params=pltpu.CompilerParams(use_tc_tiling_on_sc=False),
  )
  def spmm(sup_hbm, src_hbm, dst_hbm, out_hbm, sidx, didx, rows, acc,
           gsem0, gsem1, ssem0, ssem1):
    cid = lax.axis_index("c")
    sid = lax.axis_index("s")
    wid = sid * NC + cid
    gsem = (gsem0, gsem1)
    ssem = (ssem0, ssem1)

    def zrow(j, carry):
      for k in range(h // 16):
        rows[0, j, pl.ds(k * 16, 16)] = jnp.zeros((16,), jnp.float32)
      return carry
    lax.fori_loop(0, CH, zrow, 0)
    for z in range(zpt):
      pltpu.sync_copy(rows.at[0], acc.at[pl.ds((sid * zpt + z) * CH, CH)])
    plsc.subcore_barrier()

    ebase = wid * ept

    def idx_load(g, b):
      off = ebase + g * CH
      pltpu.sync_copy(src_hbm.at[pl.ds(off, CH)], sidx.at[b])
      pltpu.sync_copy(dst_hbm.at[pl.ds(off, CH)], didx.at[b])

    def gather_start(b):
      pltpu.async_copy(sup_hbm.at[sidx.at[b]], rows.at[b], gsem[b])

    def gather_wait(b):
      pltpu.make_async_copy(sup_hbm.at[sidx.at[b]], rows.at[b], gsem[b]).wait()

    def scatter_start(b):
      pltpu.async_copy(rows.at[b], acc.at[didx.at[b]], ssem[b], add=True)

    def scatter_wait(b):
      pltpu.make_async_copy(rows.at[b], acc.at[didx.at[b]], ssem[b]).wait()

    idx_load(0, 0)
    gather_start(0)

    def pair(p, carry):
      g0 = 2 * p
      @pl.when(p > 0)
      def _():
        scatter_wait(1)
      idx_load(g0 + 1, 1)
      gather_wait(0)
      scatter_start(0)
      gather_start(1)
      scatter_wait(0)
      @pl.when(p < n_pairs - 1)
      def _():
        idx_load(g0 + 2, 0)
      gather_wait(1)
      scatter_start(1)
      @pl.when(p < n_pairs - 1)
      def _():
        gather_start(0)
      return carry
    lax.fori_loop(0, n_pairs, pair, 0)
    scatter_wait(1)

    plsc.subcore_barrier()

    @pl.when(sid < NS - 1)
    def _():
      pltpu.sync_copy(acc.at[pl.ds(sid * rpt, rpt)],
                      out_hbm.at[cid].at[pl.ds(sid * rpt, rpt)])

    @pl.when(sid == NS - 1)
    def _():
      pltpu.sync_copy(acc.at[pl.ds((NS - 1) * rpt, rpt_last)],
                      out_hbm.at[cid].at[pl.ds((NS - 1) * rpt, rpt_last)])

  return spmm(support, src, dst)


def _mm(x, w, bm):
  n, d = x.shape
  h = w.shape[1]

  def body(x_ref, w_ref, o_ref):
    o_ref[...] = lax.dot_general(
        x_ref[...], w_ref[...], (((1,), (0,)), ((), ())),
        preferred_element_type=jnp.float32, precision=lax.Precision.HIGHEST)

  return pl.pallas_call(
      body,
      grid=(n // bm,),
      in_specs=[
          pl.BlockSpec((bm, d), lambda i: (i, 0)),
          pl.BlockSpec((d, h), lambda i: (0, 0)),
      ],
      out_specs=pl.BlockSpec((bm, h), lambda i: (i, 0)),
      out_shape=jax.ShapeDtypeStruct((n, h), jnp.float32),
  )(x, w)


def _fused_relu_mm(p, w, bm):
  _, n, d = p.shape
  h = w.shape[1]

  def body(p_ref, w_ref, o_ref):
    hid = jnp.maximum(p_ref[0] + p_ref[1], 0.0)
    o_ref[...] = lax.dot_general(
        hid, w_ref[...], (((1,), (0,)), ((), ())),
        preferred_element_type=jnp.float32, precision=lax.Precision.HIGHEST)

  return pl.pallas_call(
      body,
      grid=(n // bm,),
      in_specs=[
          pl.BlockSpec((2, bm, d), lambda i: (0, i, 0)),
          pl.BlockSpec((d, h), lambda i: (0, 0)),
      ],
      out_specs=pl.BlockSpec((bm, h), lambda i: (i, 0)),
      out_shape=jax.ShapeDtypeStruct((n, h), jnp.float32),
  )(p, w)


def _decoder(p34, h3, bm, bn):
  _, n, h2 = p34.shape

  def body(pi_ref, pj_ref, dc_ref, mu_ref, lv_ref):
    zi_full = pi_ref[0] + pi_ref[1]
    zj_full = pj_ref[0] + pj_ref[1]
    zi = zi_full[:, :h3]
    zj = zj_full[:, :h3]
    dc_ref[...] = lax.dot_general(
        zi, zj, (((1,), (1,)), ((), ())),
        preferred_element_type=jnp.float32, precision=lax.Precision.HIGHEST)

    @pl.when(pl.program_id(1) == 0)
    def _():
      mu_ref[...] = zi
      lv_ref[...] = zi_full[:, h3:]

  return pl.pallas_call(
      body,
      grid=(-(-n // bm), -(-n // bn)),
      in_specs=[
          pl.BlockSpec((2, bm, h2), lambda i, j: (0, i, 0)),
          pl.BlockSpec((2, bn, h2), lambda i, j: (0, j, 0)),
      ],
      out_specs=[
          pl.BlockSpec((bm, bn), lambda i, j: (i, j)),
          pl.BlockSpec((bm, h3), lambda i, j: (i, 0)),
          pl.BlockSpec((bm, h3), lambda i, j: (i, 0)),
      ],
      out_shape=[
          jax.ShapeDtypeStruct((n, n), jnp.float32),
          jax.ShapeDtypeStruct((n, h3), jnp.float32),
          jax.ShapeDtypeStruct((n, h3), jnp.float32),
      ],
  )(p34, p34)


@jax.jit
def kernel(x, edge_index, W1, W2, W3, W4):
  n, _ = x.shape
  e = edge_index.shape[1]

  src = edge_index[0].astype(jnp.int32)
  dst = edge_index[1].astype(jnp.int32)
  e_pad = -(-e // (NW * CH * 2)) * (NW * CH * 2)
  if e_pad != e:
    pad = e_pad - e
    src = jnp.concatenate([src, jnp.zeros((pad,), jnp.int32)])
    dst = jnp.concatenate([dst, jnp.full((pad,), n, jnp.int32)])

  support1 = _mm(x, W1, bm=2000)
  p1 = _spmm_partials(support1, src, dst, n)
  support2 = _fused_relu_mm(p1, W2, bm=2000)
  p2 = _spmm_partials(support2, src, dst, n)
  w34 = jnp.concatenate([W3, W4], axis=1)
  support34 = _fused_relu_mm(p2, w34, bm=2000)
  p34 = _spmm_partials(support34, src, dst, n)
  dc, mu, logvar = _decoder(p34, W3.shape[1], bm=1000, bn=2048)
  return (dc, mu, logvar)


# speedup vs baseline: 6.7509x; 1.3812x over previous
import functools

import jax
import jax.numpy as jnp
from jax import lax
from jax.experimental import pallas as pl
from jax.experimental.pallas import tpu as pltpu
from jax.experimental.pallas import tpu_sc as plsc

NC = 2
NS = 16
NW = NC * NS
CH = 128


def _spmm_partials(support, src, dst, n_rows):
  e_pad = src.shape[0]
  h = support.shape[1]
  ept = e_pad // NW
  n_chunks = ept // CH
  acc_rows = -(-(n_rows + 1) // (NS * CH)) * (NS * CH)
  zpt = acc_rows // (NS * CH)
  rpt = ((n_rows + NS - 1) // NS + 7) // 8 * 8
  rpt_last = n_rows - rpt * (NS - 1)
  assert rpt_last > 0

  mesh = plsc.VectorSubcoreMesh(core_axis_name="c", subcore_axis_name="s")

  NB = 8
  K = 4
  assert n_chunks % NB == 0
  n_rounds = n_chunks // NB

  @functools.partial(
      pl.kernel,
      out_type=jax.ShapeDtypeStruct((NC, n_rows, h), jnp.float32),
      mesh=mesh,
      scratch_types=(
          [
              pltpu.VMEM((NB, CH), jnp.int32),
              pltpu.VMEM((NB, CH), jnp.int32),
              pltpu.VMEM((NB, CH, h), jnp.float32),
              pltpu.VMEM_SHARED((acc_rows, h), jnp.float32),
          ]
          + [pltpu.SemaphoreType.DMA] * (3 * NB)
      ),
      compiler_params=pltpu.CompilerParams(use_tc_tiling_on_sc=False),
  )
  def spmm(sup_hbm, src_hbm, dst_hbm, out_hbm, sidx, didx, rows, acc, *sems):
    isem = sems[0:NB]
    gsem = sems[NB:2 * NB]
    ssem = sems[2 * NB:3 * NB]
    cid = lax.axis_index("c")
    sid = lax.axis_index("s")
    wid = sid * NC + cid

    def zrow(j, carry):
      for k in range(h // 16):
        rows[0, j, pl.ds(k * 16, 16)] = jnp.zeros((16,), jnp.float32)
      return carry
    lax.fori_loop(0, CH, zrow, 0)
    for z in range(zpt):
      pltpu.sync_copy(rows.at[0], acc.at[pl.ds((sid * zpt + z) * CH, CH)])
    plsc.subcore_barrier()

    ebase = wid * ept

    def idx_start(g, b):
      off = ebase + g * CH
      pltpu.async_copy(src_hbm.at[pl.ds(off, CH)], sidx.at[b], isem[b])
      pltpu.async_copy(dst_hbm.at[pl.ds(off, CH)], didx.at[b], isem[b])

    def idx_wait(g, b):
      off = ebase + g * CH
      pltpu.make_async_copy(src_hbm.at[pl.ds(off, CH)], sidx.at[b], isem[b]).wait()
      pltpu.make_async_copy(dst_hbm.at[pl.ds(off, CH)], didx.at[b], isem[b]).wait()

    def gather_start(b):
      pltpu.async_copy(sup_hbm.at[sidx.at[b]], rows.at[b], gsem[b])

    def gather_wait(b):
      pltpu.make_async_copy(sup_hbm.at[sidx.at[b]], rows.at[b], gsem[b]).wait()

    def scatter_start(b):
      pltpu.async_copy(rows.at[b], acc.at[didx.at[b]], ssem[b], add=True)

    def scatter_wait(b):
      pltpu.make_async_copy(rows.at[b], acc.at[didx.at[b]], ssem[b]).wait()

    for f in range(K):
      idx_start(f, f)
      idx_wait(f, f)
      gather_start(f)
    idx_start(K, K)

    def round_body(r, carry):
      for u in range(NB):
        g = r * NB + u
        gather_wait(u)
        scatter_start(u)
        fi = g + K + 1
        si = (u + K + 1) % NB

        @pl.when(fi < n_chunks)
        def _():
          @pl.when(fi >= NB)
          def _():
            scatter_wait(si)
          idx_start(fi, si)

        fg = g + K
        sg = (u + K) % NB

        @pl.when(fg < n_chunks)
        def _():
          idx_wait(fg, sg)
          gather_start(sg)
      return carry
    lax.fori_loop(0, n_rounds, round_body, 0)
    for u in range(NB):
      scatter_wait(u)

    plsc.subcore_barrier()

    @pl.when(sid < NS - 1)
    def _():
      pltpu.sync_copy(acc.at[pl.ds(sid * rpt, rpt)],
                      out_hbm.at[cid].at[pl.ds(sid * rpt, rpt)])

    @pl.when(sid == NS - 1)
    def _():
      pltpu.sync_copy(acc.at[pl.ds((NS - 1) * rpt, rpt_last)],
                      out_hbm.at[cid].at[pl.ds((NS - 1) * rpt, rpt_last)])

  return spmm(support, src, dst)


def _mm(x, w, bm):
  n, d = x.shape
  h = w.shape[1]

  def body(x_ref, w_ref, o_ref):
    o_ref[...] = lax.dot_general(
        x_ref[...], w_ref[...], (((1,), (0,)), ((), ())),
        preferred_element_type=jnp.float32, precision=lax.Precision.HIGHEST)

  return pl.pallas_call(
      body,
      grid=(n // bm,),
      in_specs=[
          pl.BlockSpec((bm, d), lambda i: (i, 0)),
          pl.BlockSpec((d, h), lambda i: (0, 0)),
      ],
      out_specs=pl.BlockSpec((bm, h), lambda i: (i, 0)),
      out_shape=jax.ShapeDtypeStruct((n, h), jnp.float32),
  )(x, w)


def _fused_relu_mm(p, w, bm):
  _, n, d = p.shape
  h = w.shape[1]

  def body(p_ref, w_ref, o_ref):
    hid = jnp.maximum(p_ref[0] + p_ref[1], 0.0)
    o_ref[...] = lax.dot_general(
        hid, w_ref[...], (((1,), (0,)), ((), ())),
        preferred_element_type=jnp.float32, precision=lax.Precision.HIGHEST)

  return pl.pallas_call(
      body,
      grid=(n // bm,),
      in_specs=[
          pl.BlockSpec((2, bm, d), lambda i: (0, i, 0)),
          pl.BlockSpec((d, h), lambda i: (0, 0)),
      ],
      out_specs=pl.BlockSpec((bm, h), lambda i: (i, 0)),
      out_shape=jax.ShapeDtypeStruct((n, h), jnp.float32),
  )(p, w)


def _decoder(p34, h3, bm, bn):
  _, n, h2 = p34.shape

  def body(pi_ref, pj_ref, dc_ref, mu_ref, lv_ref):
    zi_full = pi_ref[0] + pi_ref[1]
    zj_full = pj_ref[0] + pj_ref[1]
    zi = zi_full[:, :h3]
    zj = zj_full[:, :h3]
    dc_ref[...] = lax.dot_general(
        zi, zj, (((1,), (1,)), ((), ())),
        preferred_element_type=jnp.float32)

    @pl.when(pl.program_id(1) == 0)
    def _():
      mu_ref[...] = zi
      lv_ref[...] = zi_full[:, h3:]

  return pl.pallas_call(
      body,
      grid=(-(-n // bm), -(-n // bn)),
      in_specs=[
          pl.BlockSpec((2, bm, h2), lambda i, j: (0, i, 0)),
          pl.BlockSpec((2, bn, h2), lambda i, j: (0, j, 0)),
      ],
      out_specs=[
          pl.BlockSpec((bm, bn), lambda i, j: (i, j)),
          pl.BlockSpec((bm, h3), lambda i, j: (i, 0)),
          pl.BlockSpec((bm, h3), lambda i, j: (i, 0)),
      ],
      out_shape=[
          jax.ShapeDtypeStruct((n, n), jnp.float32),
          jax.ShapeDtypeStruct((n, h3), jnp.float32),
          jax.ShapeDtypeStruct((n, h3), jnp.float32),
      ],
  )(p34, p34)


@jax.jit
def kernel(x, edge_index, W1, W2, W3, W4):
  n, _ = x.shape
  e = edge_index.shape[1]

  src = edge_index[0].astype(jnp.int32)
  dst = edge_index[1].astype(jnp.int32)
  e_pad = -(-e // (NW * CH * 8)) * (NW * CH * 8)
  if e_pad != e:
    pad = e_pad - e
    src = jnp.concatenate([src, jnp.zeros((pad,), jnp.int32)])
    dst = jnp.concatenate([dst, jnp.full((pad,), n, jnp.int32)])

  support1 = _mm(x, W1, bm=2000)
  p1 = _spmm_partials(support1, src, dst, n)
  support2 = _fused_relu_mm(p1, W2, bm=2000)
  p2 = _spmm_partials(support2, src, dst, n)
  w34 = jnp.concatenate([W3, W4], axis=1)
  support34 = _fused_relu_mm(p2, w34, bm=2000)
  p34 = _spmm_partials(support34, src, dst, n)
  dc, mu, logvar = _decoder(p34, W3.shape[1], bm=1000, bn=2048)
  return (dc, mu, logvar)


# speedup vs baseline: 11.0831x; 1.6417x over previous
import functools

import jax
import jax.numpy as jnp
from jax import lax
from jax.experimental import pallas as pl
from jax.experimental.pallas import tpu as pltpu
from jax.experimental.pallas import tpu_sc as plsc

NC = 2
NS = 16
CH = 128
NB = 8
KP = 4


def _spmm_colsplit(support, src, dst, n_rows):
  e_pad = src.shape[0]
  hc = support.shape[2]
  ept = e_pad // NS
  n_chunks = ept // CH
  assert n_chunks % NB == 0
  n_rounds = n_chunks // NB
  acc_rows = -(-(n_rows + 1) // (NS * CH)) * (NS * CH)
  zpt = acc_rows // (NS * CH)
  rpt = ((n_rows + NS - 1) // NS + 7) // 8 * 8
  rpt_last = n_rows - rpt * (NS - 1)
  assert rpt_last > 0

  mesh = plsc.VectorSubcoreMesh(core_axis_name="c", subcore_axis_name="s")

  @functools.partial(
      pl.kernel,
      out_type=jax.ShapeDtypeStruct((NC, n_rows, hc), jnp.float32),
      mesh=mesh,
      scratch_types=(
          [
              pltpu.VMEM((NB, CH), jnp.int32),
              pltpu.VMEM((NB, CH), jnp.int32),
              pltpu.VMEM((NB, CH, hc), jnp.float32),
              pltpu.VMEM_SHARED((acc_rows, hc), jnp.float32),
              pltpu.VMEM_SHARED((acc_rows, hc), jnp.float32),
          ]
          + [pltpu.SemaphoreType.DMA] * (3 * NB)
      ),
      compiler_params=pltpu.CompilerParams(use_tc_tiling_on_sc=False),
  )
  def spmm(sup_hbm, src_hbm, dst_hbm, out_hbm, sidx, didx, rows, acc, sup,
           *sems):
    isem = sems[0:NB]
    gsem = sems[NB:2 * NB]
    ssem = sems[2 * NB:3 * NB]
    cid = lax.axis_index("c")
    sid = lax.axis_index("s")

    @pl.when(sid < NS - 1)
    def _():
      pltpu.sync_copy(sup_hbm.at[cid].at[pl.ds(sid * rpt, rpt)],
                      sup.at[pl.ds(sid * rpt, rpt)])

    @pl.when(sid == NS - 1)
    def _():
      pltpu.sync_copy(sup_hbm.at[cid].at[pl.ds((NS - 1) * rpt, rpt_last)],
                      sup.at[pl.ds((NS - 1) * rpt, rpt_last)])

    def zrow(j, carry):
      for k in range(hc // 16):
        rows[0, j, pl.ds(k * 16, 16)] = jnp.zeros((16,), jnp.float32)
      return carry
    lax.fori_loop(0, CH, zrow, 0)
    for z in range(zpt):
      pltpu.sync_copy(rows.at[0], acc.at[pl.ds((sid * zpt + z) * CH, CH)])
    plsc.subcore_barrier()

    ebase = sid * ept

    def idx_start(g, b):
      off = ebase + g * CH
      pltpu.async_copy(src_hbm.at[pl.ds(off, CH)], sidx.at[b], isem[b])
      pltpu.async_copy(dst_hbm.at[pl.ds(off, CH)], didx.at[b], isem[b])

    def idx_wait(g, b):
      off = ebase + g * CH
      pltpu.make_async_copy(src_hbm.at[pl.ds(off, CH)], sidx.at[b], isem[b]).wait()
      pltpu.make_async_copy(dst_hbm.at[pl.ds(off, CH)], didx.at[b], isem[b]).wait()

    def gather_start(b):
      pltpu.async_copy(sup.at[sidx.at[b]], rows.at[b], gsem[b])

    def gather_wait(b):
      pltpu.make_async_copy(sup.at[sidx.at[b]], rows.at[b], gsem[b]).wait()

    def scatter_start(b):
      pltpu.async_copy(rows.at[b], acc.at[didx.at[b]], ssem[b], add=True)

    def scatter_wait(b):
      pltpu.make_async_copy(rows.at[b], acc.at[didx.at[b]], ssem[b]).wait()

    for f in range(KP):
      idx_start(f, f)
      idx_wait(f, f)
      gather_start(f)
    idx_start(KP, KP)

    def round_body(r, carry):
      for u in range(NB):
        g = r * NB + u
        gather_wait(u)
        scatter_start(u)
        fi = g + KP + 1
        si = (u + KP + 1) % NB

        @pl.when(fi < n_chunks)
        def _():
          @pl.when(fi >= NB)
          def _():
            scatter_wait(si)
          idx_start(fi, si)

        fg = g + KP
        sg = (u + KP) % NB

        @pl.when(fg < n_chunks)
        def _():
          idx_wait(fg, sg)
          gather_start(sg)
      return carry
    lax.fori_loop(0, n_rounds, round_body, 0)
    for u in range(NB):
      scatter_wait(u)

    plsc.subcore_barrier()

    @pl.when(sid < NS - 1)
    def _():
      pltpu.sync_copy(acc.at[pl.ds(sid * rpt, rpt)],
                      out_hbm.at[cid].at[pl.ds(sid * rpt, rpt)])

    @pl.when(sid == NS - 1)
    def _():
      pltpu.sync_copy(acc.at[pl.ds((NS - 1) * rpt, rpt_last)],
                      out_hbm.at[cid].at[pl.ds((NS - 1) * rpt, rpt_last)])

  return spmm(support, src, dst)


def _mm_split(x, w2, bm):
  n, d = x.shape
  _, _, hc = w2.shape

  def body(x_ref, w_ref, o_ref):
    o_ref[0] = lax.dot_general(
        x_ref[...], w_ref[0], (((1,), (0,)), ((), ())),
        preferred_element_type=jnp.float32, precision=lax.Precision.HIGHEST)

  return pl.pallas_call(
      body,
      grid=(2, n // bm),
      in_specs=[
          pl.BlockSpec((bm, d), lambda c, i: (i, 0)),
          pl.BlockSpec((1, d, hc), lambda c, i: (c, 0, 0)),
      ],
      out_specs=pl.BlockSpec((1, bm, hc), lambda c, i: (c, i, 0)),
      out_shape=jax.ShapeDtypeStruct((2, n, hc), jnp.float32),
  )(x, w2)


def _fused_relu_mm_split(p, w2, bm):
  _, n, hcin = p.shape
  _, _, hc = w2.shape

  def body(p_ref, w_ref, o_ref):
    hid = jnp.maximum(
        jnp.concatenate([p_ref[0], p_ref[1]], axis=1), 0.0)
    o_ref[0] = lax.dot_general(
        hid, w_ref[0], (((1,), (0,)), ((), ())),
        preferred_element_type=jnp.float32, precision=lax.Precision.HIGHEST)

  return pl.pallas_call(
      body,
      grid=(2, n // bm),
      in_specs=[
          pl.BlockSpec((2, bm, hcin), lambda c, i: (0, i, 0)),
          pl.BlockSpec((1, 2 * hcin, hc), lambda c, i: (c, 0, 0)),
      ],
      out_specs=pl.BlockSpec((1, bm, hc), lambda c, i: (c, i, 0)),
      out_shape=jax.ShapeDtypeStruct((2, n, hc), jnp.float32),
  )(p, w2)


def _decoder_mm(mu, bm, bn):
  n, h3 = mu.shape

  def body(zi_ref, zj_ref, dc_ref):
    dc_ref[...] = lax.dot_general(
        zi_ref[...], zj_ref[...], (((1,), (1,)), ((), ())),
        preferred_element_type=jnp.float32)

  return pl.pallas_call(
      body,
      grid=(-(-n // bm), -(-n // bn)),
      in_specs=[
          pl.BlockSpec((bm, h3), lambda i, j: (i, 0)),
          pl.BlockSpec((bn, h3), lambda i, j: (j, 0)),
      ],
      out_specs=pl.BlockSpec((bm, bn), lambda i, j: (i, j)),
      out_shape=jax.ShapeDtypeStruct((n, n), jnp.float32),
  )(mu, mu)


@jax.jit
def kernel(x, edge_index, W1, W2, W3, W4):
  n, _ = x.shape
  e = edge_index.shape[1]

  src = edge_index[0].astype(jnp.int32)
  dst = edge_index[1].astype(jnp.int32)
  quantum = NS * CH * NB
  e_pad = -(-e // quantum) * quantum
  if e_pad != e:
    pad = e_pad - e
    src = jnp.concatenate([src, jnp.zeros((pad,), jnp.int32)])
    dst = jnp.concatenate([dst, jnp.full((pad,), n, jnp.int32)])

  def colsplit(w):
    d, h = w.shape
    return w.reshape(d, 2, h // 2).transpose(1, 0, 2)

  support1 = _mm_split(x, colsplit(W1), bm=2000)
  p1 = _spmm_colsplit(support1, src, dst, n)
  support2 = _fused_relu_mm_split(p1, colsplit(W2), bm=2000)
  p2 = _spmm_colsplit(support2, src, dst, n)
  w34 = jnp.stack([W3, W4])
  support34 = _fused_relu_mm_split(p2, w34, bm=2000)
  p34 = _spmm_colsplit(support34, src, dst, n)
  mu = p34[0]
  logvar = p34[1]
  dc = _decoder_mm(mu, bm=1000, bn=2048)
  return (dc, mu, logvar)


# speedup vs baseline: 11.9847x; 1.0813x over previous
import functools

import jax
import jax.numpy as jnp
from jax import lax
from jax.experimental import pallas as pl
from jax.experimental.pallas import tpu as pltpu
from jax.experimental.pallas import tpu_sc as plsc

NC = 2
NS = 16
CH = 128
NB = 8
KP = 4


def _spmm_colsplit(sup0, sup1, edges, n_rows):
  e_pad = edges.shape[1]
  hc = sup0.shape[1]
  ept = e_pad // NS
  n_chunks = ept // CH
  assert n_chunks % NB == 0
  n_rounds = n_chunks // NB
  acc_rows = -(-(n_rows + 1) // (NS * CH)) * (NS * CH)
  zpt = acc_rows // (NS * CH)
  rpt = ((n_rows + NS - 1) // NS + 7) // 8 * 8
  rpt_last = n_rows - rpt * (NS - 1)
  assert rpt_last > 0

  mesh = plsc.VectorSubcoreMesh(core_axis_name="c", subcore_axis_name="s")

  @functools.partial(
      pl.kernel,
      out_type=(jax.ShapeDtypeStruct((n_rows, hc), jnp.float32),
                jax.ShapeDtypeStruct((n_rows, hc), jnp.float32)),
      mesh=mesh,
      scratch_types=(
          [
              pltpu.VMEM((NB, CH), jnp.int32),
              pltpu.VMEM((NB, CH), jnp.int32),
              pltpu.VMEM((NB, CH, hc), jnp.float32),
              pltpu.VMEM_SHARED((acc_rows, hc), jnp.float32),
              pltpu.VMEM_SHARED((acc_rows, hc), jnp.float32),
          ]
          + [pltpu.SemaphoreType.DMA] * (3 * NB)
      ),
      compiler_params=pltpu.CompilerParams(use_tc_tiling_on_sc=False),
  )
  def spmm(sup0_hbm, sup1_hbm, edges_hbm, out0_hbm, out1_hbm,
           sidx, didx, rows, acc, sup, *sems):
    isem = sems[0:NB]
    gsem = sems[NB:2 * NB]
    ssem = sems[2 * NB:3 * NB]
    cid = lax.axis_index("c")
    sid = lax.axis_index("s")
    src_hbm = edges_hbm.at[0]
    dst_hbm = edges_hbm.at[1]

    def zrow(j, carry):
      for k in range(hc // 16):
        rows[0, j, pl.ds(k * 16, 16)] = jnp.zeros((16,), jnp.float32)
      return carry
    lax.fori_loop(0, CH, zrow, 0)

    def stage(sup_hbm):
      @pl.when(sid < NS - 1)
      def _():
        pltpu.sync_copy(sup_hbm.at[pl.ds(sid * rpt, rpt)],
                        sup.at[pl.ds(sid * rpt, rpt)])

      @pl.when(sid == NS - 1)
      def _():
        pltpu.sync_copy(sup_hbm.at[pl.ds((NS - 1) * rpt, rpt_last)],
                        sup.at[pl.ds((NS - 1) * rpt, rpt_last)])
        pltpu.sync_copy(rows.at[0].at[pl.ds(0, 8)],
                        sup.at[pl.ds(n_rows, 8)])

    @pl.when(cid == 0)
    def _():
      stage(sup0_hbm)

    @pl.when(cid == 1)
    def _():
      stage(sup1_hbm)

    for z in range(zpt):
      pltpu.sync_copy(rows.at[0], acc.at[pl.ds((sid * zpt + z) * CH, CH)])
    plsc.subcore_barrier()

    ebase = sid * ept

    def idx_start(g, b):
      off = ebase + g * CH
      pltpu.async_copy(src_hbm.at[pl.ds(off, CH)], sidx.at[b], isem[b])
      pltpu.async_copy(dst_hbm.at[pl.ds(off, CH)], didx.at[b], isem[b])

    def idx_wait(g, b):
      off = ebase + g * CH
      pltpu.make_async_copy(src_hbm.at[pl.ds(off, CH)], sidx.at[b], isem[b]).wait()
      pltpu.make_async_copy(dst_hbm.at[pl.ds(off, CH)], didx.at[b], isem[b]).wait()

    def gather_start(b):
      pltpu.async_copy(sup.at[sidx.at[b]], rows.at[b], gsem[b])

    def gather_wait(b):
      pltpu.make_async_copy(sup.at[sidx.at[b]], rows.at[b], gsem[b]).wait()

    def scatter_start(b):
      pltpu.async_copy(rows.at[b], acc.at[didx.at[b]], ssem[b], add=True)

    def scatter_wait(b):
      pltpu.make_async_copy(rows.at[b], acc.at[didx.at[b]], ssem[b]).wait()

    for f in range(KP):
      idx_start(f, f)
      idx_wait(f, f)
      gather_start(f)
    idx_start(KP, KP)

    def round_body(r, carry):
      for u in range(NB):
        g = r * NB + u
        gather_wait(u)
        scatter_start(u)
        fi = g + KP + 1
        si = (u + KP + 1) % NB

        @pl.when(fi < n_chunks)
        def _():
          @pl.when(fi >= NB)
          def _():
            scatter_wait(si)
          idx_start(fi, si)

        fg = g + KP
        sg = (u + KP) % NB

        @pl.when(fg < n_chunks)
        def _():
          idx_wait(fg, sg)
          gather_start(sg)
      return carry
    lax.fori_loop(0, n_rounds, round_body, 0)
    for u in range(NB):
      scatter_wait(u)

    plsc.subcore_barrier()

    def drain(out_hbm):
      @pl.when(sid < NS - 1)
      def _():
        pltpu.sync_copy(acc.at[pl.ds(sid * rpt, rpt)],
                        out_hbm.at[pl.ds(sid * rpt, rpt)])

      @pl.when(sid == NS - 1)
      def _():
        pltpu.sync_copy(acc.at[pl.ds((NS - 1) * rpt, rpt_last)],
                        out_hbm.at[pl.ds((NS - 1) * rpt, rpt_last)])

    @pl.when(cid == 0)
    def _():
      drain(out0_hbm)

    @pl.when(cid == 1)
    def _():
      drain(out1_hbm)

  return spmm(sup0, sup1, edges)


def _mm_split(x, w, bm):
  n, d = x.shape
  h = w.shape[1]
  hc = h // 2

  def body(x_ref, w_ref, o0_ref, o1_ref):
    full = lax.dot_general(
        x_ref[...], w_ref[...], (((1,), (0,)), ((), ())),
        preferred_element_type=jnp.float32, precision=lax.Precision.HIGHEST)
    o0_ref[...] = full[:, :hc]
    o1_ref[...] = full[:, hc:]

  return pl.pallas_call(
      body,
      grid=(n // bm,),
      in_specs=[
          pl.BlockSpec((bm, d), lambda i: (i, 0)),
          pl.BlockSpec((d, h), lambda i: (0, 0)),
      ],
      out_specs=[
          pl.BlockSpec((bm, hc), lambda i: (i, 0)),
          pl.BlockSpec((bm, hc), lambda i: (i, 0)),
      ],
      out_shape=[
          jax.ShapeDtypeStruct((n, hc), jnp.float32),
          jax.ShapeDtypeStruct((n, hc), jnp.float32),
      ],
  )(x, w)


def _fused_relu_mm_split(p0, p1, ws, bm):
  n, hcin = p0.shape
  h = sum(w.shape[1] for w in ws)
  hc = h // 2

  def body(p0_ref, p1_ref, *refs):
    w_refs = refs[:len(ws)]
    o0_ref, o1_ref = refs[len(ws):]
    hid = jnp.maximum(
        jnp.concatenate([p0_ref[...], p1_ref[...]], axis=1), 0.0)
    parts = [
        lax.dot_general(
            hid, w_ref[...], (((1,), (0,)), ((), ())),
            preferred_element_type=jnp.float32,
            precision=lax.Precision.HIGHEST)
        for w_ref in w_refs
    ]
    full = parts[0] if len(parts) == 1 else jnp.concatenate(parts, axis=1)
    o0_ref[...] = full[:, :hc]
    o1_ref[...] = full[:, hc:]

  return pl.pallas_call(
      body,
      grid=(n // bm,),
      in_specs=[
          pl.BlockSpec((bm, hcin), lambda i: (i, 0)),
          pl.BlockSpec((bm, hcin), lambda i: (i, 0)),
      ] + [
          pl.BlockSpec(w.shape, lambda i: (0, 0)) for w in ws
      ],
      out_specs=[
          pl.BlockSpec((bm, hc), lambda i: (i, 0)),
          pl.BlockSpec((bm, hc), lambda i: (i, 0)),
      ],
      out_shape=[
          jax.ShapeDtypeStruct((n, hc), jnp.float32),
          jax.ShapeDtypeStruct((n, hc), jnp.float32),
      ],
  )(p0, p1, *ws)


def _decoder_mm(mu, bm, bn):
  n, h3 = mu.shape

  def body(zi_ref, zj_ref, dc_ref):
    dc_ref[...] = lax.dot_general(
        zi_ref[...], zj_ref[...], (((1,), (1,)), ((), ())),
        preferred_element_type=jnp.float32)

  return pl.pallas_call(
      body,
      grid=(-(-n // bm), -(-n // bn)),
      in_specs=[
          pl.BlockSpec((bm, h3), lambda i, j: (i, 0)),
          pl.BlockSpec((bn, h3), lambda i, j: (j, 0)),
      ],
      out_specs=pl.BlockSpec((bm, bn), lambda i, j: (i, j)),
      out_shape=jax.ShapeDtypeStruct((n, n), jnp.float32),
  )(mu, mu)


@jax.jit
def kernel(x, edge_index, W1, W2, W3, W4):
  n, _ = x.shape
  e = edge_index.shape[1]

  quantum = NS * CH * NB
  e_pad = -(-e // quantum) * quantum
  edges = edge_index.astype(jnp.int32)
  if e_pad != e:
    edges = jnp.pad(edges, ((0, 0), (0, e_pad - e)), constant_values=n)

  s1a, s1b = _mm_split(x, W1, bm=2000)
  p1a, p1b = _spmm_colsplit(s1a, s1b, edges, n)
  s2a, s2b = _fused_relu_mm_split(p1a, p1b, (W2,), bm=2000)
  p2a, p2b = _spmm_colsplit(s2a, s2b, edges, n)
  s3a, s3b = _fused_relu_mm_split(p2a, p2b, (W3, W4), bm=2000)
  mu, logvar = _spmm_colsplit(s3a, s3b, edges, n)
  dc = _decoder_mm(mu, bm=1000, bn=2048)
  return (dc, mu, logvar)


# speedup vs baseline: 12.3854x; 1.0334x over previous
import functools

import jax
import jax.numpy as jnp
from jax import lax
from jax.experimental import pallas as pl
from jax.experimental.pallas import tpu as pltpu
from jax.experimental.pallas import tpu_sc as plsc

NC = 2
NS = 16
CH = 128
NB = 8
KP = 4


def _spmm_colsplit(sup0, sup1, edges, n_rows):
  e_pad = edges.shape[1]
  hc = sup0.shape[1]
  ept = e_pad // NS
  n_chunks = ept // CH
  assert n_chunks % NB == 0
  n_rounds = n_chunks // NB
  acc_rows = -(-(n_rows + 1) // (NS * CH)) * (NS * CH)
  zpt = acc_rows // (NS * CH)
  rpt = ((n_rows + NS - 1) // NS + 7) // 8 * 8
  rpt_last = n_rows - rpt * (NS - 1)
  assert rpt_last > 0

  mesh = plsc.VectorSubcoreMesh(core_axis_name="c", subcore_axis_name="s")

  @functools.partial(
      pl.kernel,
      out_type=(jax.ShapeDtypeStruct((n_rows, hc), jnp.float32),
                jax.ShapeDtypeStruct((n_rows, hc), jnp.float32)),
      mesh=mesh,
      scratch_types=(
          [
              pltpu.VMEM((NB, CH), jnp.int32),
              pltpu.VMEM((NB, CH), jnp.int32),
              pltpu.VMEM((NB, CH, hc), jnp.float32),
              pltpu.VMEM_SHARED((acc_rows, hc), jnp.float32),
              pltpu.VMEM_SHARED((acc_rows, hc), jnp.float32),
          ]
          + [pltpu.SemaphoreType.DMA] * (3 * NB)
      ),
      compiler_params=pltpu.CompilerParams(use_tc_tiling_on_sc=False),
  )
  def spmm(sup0_hbm, sup1_hbm, edges_hbm, out0_hbm, out1_hbm,
           sidx, didx, rows, acc, sup, *sems):
    isem = sems[0:NB]
    gsem = sems[NB:2 * NB]
    ssem = sems[2 * NB:3 * NB]
    cid = lax.axis_index("c")
    sid = lax.axis_index("s")
    src_hbm = edges_hbm.at[0]
    dst_hbm = edges_hbm.at[1]

    def zrow(j, carry):
      for k in range(hc // 16):
        rows[0, j, pl.ds(k * 16, 16)] = jnp.zeros((16,), jnp.float32)
      return carry
    lax.fori_loop(0, CH, zrow, 0)

    def stage(sup_hbm):
      @pl.when(sid < NS - 1)
      def _():
        pltpu.sync_copy(sup_hbm.at[pl.ds(sid * rpt, rpt)],
                        sup.at[pl.ds(sid * rpt, rpt)])

      @pl.when(sid == NS - 1)
      def _():
        pltpu.sync_copy(sup_hbm.at[pl.ds((NS - 1) * rpt, rpt_last)],
                        sup.at[pl.ds((NS - 1) * rpt, rpt_last)])
        pltpu.sync_copy(rows.at[0].at[pl.ds(0, 8)],
                        sup.at[pl.ds(n_rows, 8)])

    @pl.when(cid == 0)
    def _():
      stage(sup0_hbm)

    @pl.when(cid == 1)
    def _():
      stage(sup1_hbm)

    for z in range(zpt):
      pltpu.sync_copy(rows.at[0], acc.at[pl.ds((sid * zpt + z) * CH, CH)])
    plsc.subcore_barrier()

    ebase = sid * ept

    def idx_start(g, b):
      off = ebase + g * CH
      pltpu.async_copy(src_hbm.at[pl.ds(off, CH)], sidx.at[b], isem[b])
      pltpu.async_copy(dst_hbm.at[pl.ds(off, CH)], didx.at[b], isem[b])

    def idx_wait(g, b):
      off = ebase + g * CH
      pltpu.make_async_copy(src_hbm.at[pl.ds(off, CH)], sidx.at[b], isem[b]).wait()
      pltpu.make_async_copy(dst_hbm.at[pl.ds(off, CH)], didx.at[b], isem[b]).wait()

    def gather_start(b):
      pltpu.async_copy(sup.at[sidx.at[b]], rows.at[b], gsem[b])

    def gather_wait(b):
      pltpu.make_async_copy(sup.at[sidx.at[b]], rows.at[b], gsem[b]).wait()

    def scatter_start(b):
      pltpu.async_copy(rows.at[b], acc.at[didx.at[b]], ssem[b], add=True)

    def scatter_wait(b):
      pltpu.make_async_copy(rows.at[b], acc.at[didx.at[b]], ssem[b]).wait()

    for f in range(KP):
      idx_start(f, f)
      idx_wait(f, f)
      gather_start(f)
    idx_start(KP, KP)

    def round_body(r, carry):
      for u in range(NB):
        g = r * NB + u
        gather_wait(u)
        scatter_start(u)
        fi = g + KP + 1
        si = (u + KP + 1) % NB

        @pl.when(fi < n_chunks)
        def _():
          @pl.when(fi >= NB)
          def _():
            scatter_wait(si)
          idx_start(fi, si)

        fg = g + KP
        sg = (u + KP) % NB

        @pl.when(fg < n_chunks)
        def _():
          idx_wait(fg, sg)
          gather_start(sg)
      return carry
    lax.fori_loop(0, n_rounds, round_body, 0)
    for u in range(NB):
      scatter_wait(u)

    plsc.subcore_barrier()

    def drain(out_hbm):
      @pl.when(sid < NS - 1)
      def _():
        pltpu.sync_copy(acc.at[pl.ds(sid * rpt, rpt)],
                        out_hbm.at[pl.ds(sid * rpt, rpt)])

      @pl.when(sid == NS - 1)
      def _():
        pltpu.sync_copy(acc.at[pl.ds((NS - 1) * rpt, rpt_last)],
                        out_hbm.at[pl.ds((NS - 1) * rpt, rpt_last)])

    @pl.when(cid == 0)
    def _():
      drain(out0_hbm)

    @pl.when(cid == 1)
    def _():
      drain(out1_hbm)

  return spmm(sup0, sup1, edges)


def _mm_split(x, w, bm):
  n, d = x.shape
  h = w.shape[1]
  hc = h // 2

  def body(x_ref, w_ref, o0_ref, o1_ref):
    full = lax.dot_general(
        x_ref[...], w_ref[...], (((1,), (0,)), ((), ())),
        preferred_element_type=jnp.float32, precision=lax.Precision.HIGHEST)
    o0_ref[...] = full[:, :hc]
    o1_ref[...] = full[:, hc:]

  return pl.pallas_call(
      body,
      grid=(n // bm,),
      in_specs=[
          pl.BlockSpec((bm, d), lambda i: (i, 0)),
          pl.BlockSpec((d, h), lambda i: (0, 0)),
      ],
      out_specs=[
          pl.BlockSpec((bm, hc), lambda i: (i, 0)),
          pl.BlockSpec((bm, hc), lambda i: (i, 0)),
      ],
      out_shape=[
          jax.ShapeDtypeStruct((n, hc), jnp.float32),
          jax.ShapeDtypeStruct((n, hc), jnp.float32),
      ],
  )(x, w)


def _fused_relu_mm_split(p0, p1, ws, bm):
  n, hcin = p0.shape
  h = sum(w.shape[1] for w in ws)
  hc = h // 2

  def body(p0_ref, p1_ref, *refs):
    w_refs = refs[:len(ws)]
    o0_ref, o1_ref = refs[len(ws):]
    hid = jnp.maximum(
        jnp.concatenate([p0_ref[...], p1_ref[...]], axis=1), 0.0)
    parts = [
        lax.dot_general(
            hid, w_ref[...], (((1,), (0,)), ((), ())),
            preferred_element_type=jnp.float32,
            precision=lax.Precision.HIGHEST)
        for w_ref in w_refs
    ]
    full = parts[0] if len(parts) == 1 else jnp.concatenate(parts, axis=1)
    o0_ref[...] = full[:, :hc]
    o1_ref[...] = full[:, hc:]

  return pl.pallas_call(
      body,
      grid=(n // bm,),
      in_specs=[
          pl.BlockSpec((bm, hcin), lambda i: (i, 0)),
          pl.BlockSpec((bm, hcin), lambda i: (i, 0)),
      ] + [
          pl.BlockSpec(w.shape, lambda i: (0, 0)) for w in ws
      ],
      out_specs=[
          pl.BlockSpec((bm, hc), lambda i: (i, 0)),
          pl.BlockSpec((bm, hc), lambda i: (i, 0)),
      ],
      out_shape=[
          jax.ShapeDtypeStruct((n, hc), jnp.float32),
          jax.ShapeDtypeStruct((n, hc), jnp.float32),
      ],
  )(p0, p1, *ws)


def _decoder_mm(mu, bm, bn):
  n, h3 = mu.shape

  def body(zi_ref, zj_ref, dc_ref):
    dc_ref[...] = lax.dot_general(
        zi_ref[...], zj_ref[...], (((1,), (1,)), ((), ())),
        preferred_element_type=jnp.float32)

  return pl.pallas_call(
      body,
      grid=(-(-n // bm), -(-n // bn)),
      in_specs=[
          pl.BlockSpec((bm, h3), lambda i, j: (i, 0)),
          pl.BlockSpec((bn, h3), lambda i, j: (j, 0)),
      ],
      out_specs=pl.BlockSpec((bm, bn), lambda i, j: (i, j)),
      out_shape=jax.ShapeDtypeStruct((n, n), jnp.float32),
  )(mu, mu)


@jax.jit
def kernel(x, edge_index, W1, W2, W3, W4):
  n, _ = x.shape
  e = edge_index.shape[1]

  quantum = NS * CH * NB
  e_pad = -(-e // quantum) * quantum
  edges = edge_index.astype(jnp.int32)
  if e_pad != e:
    edges = jnp.pad(edges, ((0, 0), (0, e_pad - e)), constant_values=n)

  s1a, s1b = _mm_split(x, W1, bm=2000)
  p1a, p1b = _spmm_colsplit(s1a, s1b, edges, n)
  s2a, s2b = _fused_relu_mm_split(p1a, p1b, (W2,), bm=2000)
  p2a, p2b = _spmm_colsplit(s2a, s2b, edges, n)
  s3a, s3b = _fused_relu_mm_split(p2a, p2b, (W3, W4), bm=2000)
  mu, logvar = _spmm_colsplit(s3a, s3b, edges, n)
  dc = _decoder_mm(mu, bm=2000, bn=2048)
  return (dc, mu, logvar)
